# baseline (speedup 1.0000x reference)
import dataclasses
import functools

import jax
import jax.numpy as jnp
from jax import lax
from jax.experimental import pallas as pl
from jax.experimental.pallas import tpu as pltpu
from jax.experimental.pallas import tpu_sc as plsc

N = 10000
E = 320000
D_IN = 128
D_EDGE = 16
C = 128

N_PAD = 10240
N_TILES = 32
EDGES_PER_TILE = 10240
CHUNKS = 128
CHUNK = 80
E_PAD = N_TILES * EDGES_PER_TILE

_F32 = jnp.float32


def _pre1_body(x_ref, w_ref, asv_ref, adv_ref, xl_ref, as_ref, ad_ref):
    xl = jnp.dot(x_ref[...], w_ref[...], preferred_element_type=_F32)
    xl_ref[...] = xl
    as_ref[...] = jnp.sum(xl * asv_ref[...], axis=1, keepdims=True)
    ad_ref[...] = jnp.sum(xl * adv_ref[...], axis=1, keepdims=True)


def _pre1(x_pad, w, att_src_row, att_dst_row):
    return pl.pallas_call(
        _pre1_body,
        out_shape=[
            jax.ShapeDtypeStruct((N_PAD, C), _F32),
            jax.ShapeDtypeStruct((N_PAD, 1), _F32),
            jax.ShapeDtypeStruct((N_PAD, 1), _F32),
        ],
    )(x_pad, w, att_src_row, att_dst_row)


_AE_BLK = 4096
_AE_MAIN = (E // _AE_BLK) * _AE_BLK


def _pre2_body(ea_ref, we_ref, aev_ref, ae_ref):
    ve = jnp.sum(we_ref[...] * aev_ref[...], axis=1, keepdims=True)
    ae_ref[...] = lax.dot_general(
        ve, ea_ref[...], (((0,), (1,)), ((), ())),
        preferred_element_type=_F32)


def _pre2(ea, w_edge, att_edge_row):
    n = ea.shape[0]
    return pl.pallas_call(
        _pre2_body,
        grid=(n // _AE_BLK,),
        in_specs=[
            pl.BlockSpec((_AE_BLK, D_EDGE), lambda i: (i, 0)),
            pl.BlockSpec((D_EDGE, C), lambda i: (0, 0)),
            pl.BlockSpec((1, C), lambda i: (0, 0)),
        ],
        out_specs=pl.BlockSpec((1, _AE_BLK), lambda i: (0, i)),
        out_shape=jax.ShapeDtypeStruct((1, n), _F32),
    )(ea, w_edge, att_edge_row)


def _sc_body(xl_hbm, asrc_hbm, adst_hbm, sd_hbm, ae_hbm,
             outp_hbm, denp_hbm, sp_hbm, cntp_hbm,
             out_acc, den_acc, s_acc, cnt_acc,
             sd,
             rows0, rows1, stb0, stb1, dtb0, dtb1,
             asv0, asv1, adv0, adv1, aec0, aec1,
             pc0, pc1, wmc0, wmc1, sac0, sac1, zb,
             g0, g1, sa0, sa1, sb0, sb1, se0, se1, q0, q1, r0, r1):
    cid = lax.axis_index("c")
    sid = lax.axis_index("s")
    wid = cid * 16 + sid
    base = sid * 640

    sets = (
        (rows0, stb0, dtb0, asv0, adv0, aec0, pc0, wmc0, sac0,
         g0, sa0, sb0, se0, q0, r0),
        (rows1, stb1, dtb1, asv1, adv1, aec1, pc1, wmc1, sac1,
         g1, sa1, sb1, se1, q1, r1),
    )

    mask14 = jnp.full((16,), 0x3FFF, jnp.int32)
    sh14 = jnp.full((16,), 14, jnp.int32)

    def unpack(jj, S):
        stb, dtb = S[1], S[2]
        for k in range(CHUNK // 16):
            sl = pl.ds(k * 16, 16)
            pk = sd[jj, sl]
            stb[sl] = pk & mask14
            dtb[sl] = lax.shift_right_logical(pk, sh14)

    def gfire(jj, S):
        rows, stb, dtb, asv, adv, aec = S[:6]
        g, sa, sb, se = S[9:13]
        off = wid * EDGES_PER_TILE + jj * CHUNK
        pltpu.async_copy(xl_hbm.at[stb], rows, g)
        pltpu.async_copy(asrc_hbm.at[stb], asv, sa)
        pltpu.async_copy(adst_hbm.at[dtb], adv, sb)
        pltpu.async_copy(ae_hbm.at[pl.ds(off, CHUNK)], aec, se)

    def wait_scalars(jj, S):
        rows, stb, dtb, asv, adv, aec = S[:6]
        g, sa, sb, se = S[9:13]
        off = wid * EDGES_PER_TILE + jj * CHUNK
        pltpu.make_async_copy(asrc_hbm.at[stb], asv, sa).wait()
        pltpu.make_async_copy(adst_hbm.at[dtb], adv, sb).wait()
        pltpu.make_async_copy(ae_hbm.at[pl.ds(off, CHUNK)], aec, se).wait()

    def wait_rows(jj, S):
        pltpu.make_async_copy(xl_hbm.at[S[1]], S[0], S[9]).wait()

    def fire_q(jj, S):
        dtb, pc, wmc, sac, q = S[2], S[6], S[7], S[8], S[13]
        pltpu.make_async_copy(pc, den_acc.at[dtb], q).start(add=True)
        pltpu.make_async_copy(wmc, cnt_acc.at[dtb], q).start(add=True)
        pltpu.make_async_copy(sac, s_acc.at[dtb], q).start(add=True)

    def wait_q(jj, S):
        dtb, pc, wmc, sac, q = S[2], S[6], S[7], S[8], S[13]
        pltpu.make_async_copy(pc, den_acc.at[dtb], q).wait()
        pltpu.make_async_copy(wmc, cnt_acc.at[dtb], q).wait()
        pltpu.make_async_copy(sac, s_acc.at[dtb], q).wait()

    def fire_r(jj, S):
        pltpu.make_async_copy(S[0], out_acc.at[S[2]], S[14]).start(add=True)

    def wait_r(jj, S):
        pltpu.make_async_copy(S[0], out_acc.at[S[2]], S[14]).wait()

    pltpu.sync_copy(sd_hbm.at[wid], sd)

    z16 = jnp.zeros((16,), _F32)

    @pl.loop(0, 40)
    def _zero_zb(i):
        zb[pl.ds(i * 16, 16)] = z16

    @pl.loop(0, CHUNK)
    def _zero_rows(r):
        for k in range(8):
            rows0[r, pl.ds(k * 16, 16)] = z16

    pltpu.sync_copy(zb, den_acc.at[pl.ds(base, 640)])
    pltpu.sync_copy(zb, s_acc.at[pl.ds(base, 640)])
    pltpu.sync_copy(zb, cnt_acc.at[pl.ds(base, 640)])

    @pl.loop(0, 640 // CHUNK)
    def _zero_out(i):
        pltpu.sync_copy(rows0, out_acc.at[pl.ds(base + i * CHUNK, CHUNK), :])

    plsc.subcore_barrier()

    neg = jnp.full((16,), -1e9, _F32)
    zero = jnp.zeros((16,), _F32)
    one = jnp.ones((16,), _F32)

    unpack(0, sets[0])
    gfire(0, sets[0])

    @pl.loop(0, CHUNKS, step=2)
    def _edges(j):
        for b in (0, 1):
            jj = j + b
            S = sets[b]
            T = sets[1 - b]
            nxt = jj + 1

            @pl.when(jnp.logical_and(nxt < CHUNKS, jj >= 1))
            def _wrq():
                wait_r(jj - 1, T)
                wait_q(jj - 1, T)

            @pl.when(nxt < CHUNKS)
            def _gf():
                unpack(nxt, T)
                gfire(nxt, T)

            wait_scalars(jj, S)

            rows, stb, dtb, asv_b, adv_b, aec, pc, wmc, sac = S[:9]
            for k in range(CHUNK // 16):
                sl = pl.ds(k * 16, 16)
                s16 = stb[sl]
                d16 = dtb[sl]
                ae16 = aec[sl]
                t = asv_b[sl] + adv_b[sl] + ae16
                nl = s16 != d16
                alpha = jnp.where(nl, t, neg)
                alpha = jnp.where(alpha > 0.0, alpha, alpha * 0.2)
                p = jnp.exp(alpha)
                w = jnp.where(nl, one, zero)
                pc[sl] = p
                wmc[sl] = w
                sac[sl] = ae16 * w

            fire_q(jj, S)

            wait_rows(jj, S)

            @pl.loop(0, CHUNK)
            def _scale(rr):
                ridx = jnp.full((16,), 0, jnp.int32) + rr
                pv = plsc.load_gather(pc, [ridx])
                for k in range(8):
                    sl2 = pl.ds(k * 16, 16)
                    rows[rr, sl2] = rows[rr, sl2] * pv

            fire_r(jj, S)

    wait_r(CHUNKS - 1, sets[1])
    wait_q(CHUNKS - 1, sets[1])

    plsc.subcore_barrier()

    pltpu.sync_copy(den_acc.at[pl.ds(base, 640)],
                    denp_hbm.at[cid, pl.ds(base, 640)])
    pltpu.sync_copy(s_acc.at[pl.ds(base, 640)],
                    sp_hbm.at[cid, pl.ds(base, 640)])
    pltpu.sync_copy(cnt_acc.at[pl.ds(base, 640)],
                    cntp_hbm.at[cid, pl.ds(base, 640)])

    @pl.loop(0, 640 // CHUNK)
    def _pub(i):
        pltpu.sync_copy(out_acc.at[pl.ds(base + i * CHUNK, CHUNK), :],
                        outp_hbm.at[cid, pl.ds(base + i * CHUNK, CHUNK), :])


def _sc_main(xl, a_src, a_dst, sd3, ae3):
    mesh = plsc.VectorSubcoreMesh(core_axis_name="c", subcore_axis_name="s")
    cp = pltpu.CompilerParams()
    if "needs_layout_passes" in pltpu.CompilerParams.__dataclass_fields__:
        cp = dataclasses.replace(cp, needs_layout_passes=False)
    kfn = pl.kernel(
        _sc_body,
        mesh=mesh,
        compiler_params=cp,
        out_type=[
            jax.ShapeDtypeStruct((2, N_PAD, C), _F32),
            jax.ShapeDtypeStruct((2, N_PAD), _F32),
            jax.ShapeDtypeStruct((2, N_PAD), _F32),
            jax.ShapeDtypeStruct((2, N_PAD), _F32),
        ],
        scratch_types=[
            pltpu.VMEM_SHARED((N_PAD, C), _F32),
            pltpu.VMEM_SHARED((N_PAD,), _F32),
            pltpu.VMEM_SHARED((N_PAD,), _F32),
            pltpu.VMEM_SHARED((N_PAD,), _F32),
            pltpu.VMEM((CHUNKS, CHUNK), jnp.int32),
            pltpu.VMEM((CHUNK, C), _F32),
            pltpu.VMEM((CHUNK, C), _F32),
            pltpu.VMEM((CHUNK,), jnp.int32),
            pltpu.VMEM((CHUNK,), jnp.int32),
            pltpu.VMEM((CHUNK,), jnp.int32),
            pltpu.VMEM((CHUNK,), jnp.int32),
            pltpu.VMEM((CHUNK,), _F32),
            pltpu.VMEM((CHUNK,), _F32),
            pltpu.VMEM((CHUNK,), _F32),
            pltpu.VMEM((CHUNK,), _F32),
            pltpu.VMEM((CHUNK,), _F32),
            pltpu.VMEM((CHUNK,), _F32),
            pltpu.VMEM((CHUNK,), _F32),
            pltpu.VMEM((CHUNK,), _F32),
            pltpu.VMEM((CHUNK,), _F32),
            pltpu.VMEM((CHUNK,), _F32),
            pltpu.VMEM((CHUNK,), _F32),
            pltpu.VMEM((CHUNK,), _F32),
            pltpu.VMEM((640,), _F32),
            pltpu.SemaphoreType.DMA,
            pltpu.SemaphoreType.DMA,
            pltpu.SemaphoreType.DMA,
            pltpu.SemaphoreType.DMA,
            pltpu.SemaphoreType.DMA,
            pltpu.SemaphoreType.DMA,
            pltpu.SemaphoreType.DMA,
            pltpu.SemaphoreType.DMA,
            pltpu.SemaphoreType.DMA,
            pltpu.SemaphoreType.DMA,
            pltpu.SemaphoreType.DMA,
            pltpu.SemaphoreType.DMA,
        ],
    )
    return kfn(xl, a_src, a_dst, sd3, ae3)


def _epi_body(outp_ref, denp_ref, sp_ref, cntp_ref, as_ref, ad_ref,
              xl_ref, b_ref, o_ref):
    den = denp_ref[0] + denp_ref[1]
    s = sp_ref[0] + sp_ref[1]
    cnt = cntp_ref[0] + cntp_ref[1]
    a_loop = s / jnp.maximum(cnt, 1.0)
    al = as_ref[...] + ad_ref[...] + a_loop
    al = jnp.where(al > 0.0, al, 0.2 * al)
    p_l = jnp.exp(al)
    outu = outp_ref[0] + outp_ref[1]
    o_ref[...] = (outu + p_l * xl_ref[...]) / (den + p_l + 1e-16) + b_ref[...]


_EPI_BLK = 1024


def _epilogue(outp, denp, sp, cntp, a_src, a_dst, xl, bias_row):
    nb = N_PAD // _EPI_BLK
    return pl.pallas_call(
        _epi_body,
        grid=(nb,),
        in_specs=[
            pl.BlockSpec((2, _EPI_BLK, C), lambda i: (0, i, 0)),
            pl.BlockSpec((2, _EPI_BLK, 1), lambda i: (0, i, 0)),
            pl.BlockSpec((2, _EPI_BLK, 1), lambda i: (0, i, 0)),
            pl.BlockSpec((2, _EPI_BLK, 1), lambda i: (0, i, 0)),
            pl.BlockSpec((_EPI_BLK, 1), lambda i: (i, 0)),
            pl.BlockSpec((_EPI_BLK, 1), lambda i: (i, 0)),
            pl.BlockSpec((_EPI_BLK, C), lambda i: (i, 0)),
            pl.BlockSpec((1, C), lambda i: (0, 0)),
        ],
        out_specs=pl.BlockSpec((_EPI_BLK, C), lambda i: (i, 0)),
        out_shape=jax.ShapeDtypeStruct((N_PAD, C), _F32),
    )(outp, denp, sp, cntp, a_src, a_dst, xl, bias_row)


def kernel(x, edge_index, edge_attr, W, att_src, att_dst, W_edge, att_edge,
           bias):
    x_pad = jnp.zeros((N_PAD, D_IN), _F32).at[:N].set(x)
    src = edge_index[0]
    dst = edge_index[1]
    pad = E_PAD - E
    pad_idx = jnp.arange(pad, dtype=jnp.int32) % N
    src_p = jnp.concatenate([src, pad_idx])
    dst_p = jnp.concatenate([dst, pad_idx])
    ea_tail = jnp.zeros((E_PAD - _AE_MAIN, D_EDGE), _F32).at[: E - _AE_MAIN]\
        .set(edge_attr[_AE_MAIN:])

    att_src_row = att_src.reshape(1, C)
    att_dst_row = att_dst.reshape(1, C)
    att_edge_row = att_edge.reshape(1, C)

    xl, a_src2, a_dst2 = _pre1(x_pad, W, att_src_row, att_dst_row)
    ae_main = _pre2(edge_attr[:_AE_MAIN], W_edge, att_edge_row)
    ae_tail = _pre2(ea_tail, W_edge, att_edge_row)
    ae_flat = jnp.concatenate([ae_main.reshape(-1), ae_tail.reshape(-1)])

    sd3 = (src_p | (dst_p << 14)).reshape(N_TILES, CHUNKS, CHUNK)

    outp, denp, sp, cntp = _sc_main(
        xl, a_src2.reshape(N_PAD), a_dst2.reshape(N_PAD), sd3, ae_flat)

    out = _epilogue(outp,
                    denp.reshape(2, N_PAD, 1),
                    sp.reshape(2, N_PAD, 1),
                    cntp.reshape(2, N_PAD, 1),
                    a_src2, a_dst2, xl, bias.reshape(1, C))
    return out[:N]


# speedup vs baseline: 1.4556x; 1.4556x over previous
import dataclasses
import functools

import jax
import jax.numpy as jnp
from jax import lax
from jax.experimental import pallas as pl
from jax.experimental.pallas import tpu as pltpu
from jax.experimental.pallas import tpu_sc as plsc

N = 10000
E = 320000
D_IN = 128
D_EDGE = 16
C = 128

N_PAD = 10240
N_TILES = 32
EDGES_PER_TILE = 10240
CHUNKS = 128
CHUNK = 80
E_PAD = N_TILES * EDGES_PER_TILE

_F32 = jnp.float32


def _pre1_body(x_ref, w_ref, asv_ref, adv_ref, xl_ref, as_ref, ad_ref):
    xl = jnp.dot(x_ref[...], w_ref[...], preferred_element_type=_F32)
    xl_ref[...] = xl
    as_ref[...] = jnp.sum(xl * asv_ref[...], axis=1, keepdims=True)
    ad_ref[...] = jnp.sum(xl * adv_ref[...], axis=1, keepdims=True)


def _pre1(x_pad, w, att_src_row, att_dst_row):
    return pl.pallas_call(
        _pre1_body,
        out_shape=[
            jax.ShapeDtypeStruct((N_PAD, C), _F32),
            jax.ShapeDtypeStruct((N_PAD, 1), _F32),
            jax.ShapeDtypeStruct((N_PAD, 1), _F32),
        ],
    )(x_pad, w, att_src_row, att_dst_row)


_AE_BLK = 6400


def _pre2_body(ea_ref, we_ref, aev_ref, ae_ref):
    ve = jnp.sum(we_ref[...] * aev_ref[...], axis=1, keepdims=True)
    ae_ref[...] = jnp.sum(ea_ref[...] * ve, axis=0, keepdims=True)


def _pre2(ea_t, w_edge, att_edge_row):
    n = ea_t.shape[1]
    return pl.pallas_call(
        _pre2_body,
        grid=(n // _AE_BLK,),
        in_specs=[
            pl.BlockSpec((D_EDGE, _AE_BLK), lambda i: (0, i)),
            pl.BlockSpec((D_EDGE, C), lambda i: (0, 0)),
            pl.BlockSpec((1, C), lambda i: (0, 0)),
        ],
        out_specs=pl.BlockSpec((1, _AE_BLK), lambda i: (0, i)),
        out_shape=jax.ShapeDtypeStruct((1, n), _F32),
    )(ea_t, w_edge, att_edge_row)


def _sc_body(xl_hbm, asrc_hbm, adst_hbm, sd_hbm, ae_hbm,
             outp_hbm, denp_hbm, sp_hbm, cntp_hbm,
             out_acc, den_acc, s_acc, cnt_acc,
             sd,
             rows0, rows1, stb0, stb1, dtb0, dtb1,
             asv0, asv1, adv0, adv1, aec0, aec1,
             pc0, pc1, wmc0, wmc1, sac0, sac1, zb,
             g0, g1, sa0, sa1, sb0, sb1, se0, se1, q0, q1, r0, r1):
    cid = lax.axis_index("c")
    sid = lax.axis_index("s")
    wid = cid * 16 + sid
    base = sid * 640

    sets = (
        (rows0, stb0, dtb0, asv0, adv0, aec0, pc0, wmc0, sac0,
         g0, sa0, sb0, se0, q0, r0),
        (rows1, stb1, dtb1, asv1, adv1, aec1, pc1, wmc1, sac1,
         g1, sa1, sb1, se1, q1, r1),
    )

    mask14 = jnp.full((16,), 0x3FFF, jnp.int32)
    sh14 = jnp.full((16,), 14, jnp.int32)

    def unpack(jj, S):
        stb, dtb = S[1], S[2]
        for k in range(CHUNK // 16):
            sl = pl.ds(k * 16, 16)
            pk = sd[jj, sl]
            stb[sl] = pk & mask14
            dtb[sl] = lax.shift_right_logical(pk, sh14)

    def gfire(jj, S):
        rows, stb, dtb, asv, adv, aec = S[:6]
        g, sa, sb, se = S[9:13]
        off = wid * EDGES_PER_TILE + jj * CHUNK
        pltpu.async_copy(xl_hbm.at[stb], rows, g)
        pltpu.async_copy(asrc_hbm.at[stb], asv, sa)
        pltpu.async_copy(adst_hbm.at[dtb], adv, sb)
        pltpu.async_copy(ae_hbm.at[pl.ds(off, CHUNK)], aec, se)

    def wait_scalars(jj, S):
        rows, stb, dtb, asv, adv, aec = S[:6]
        g, sa, sb, se = S[9:13]
        off = wid * EDGES_PER_TILE + jj * CHUNK
        pltpu.make_async_copy(asrc_hbm.at[stb], asv, sa).wait()
        pltpu.make_async_copy(adst_hbm.at[dtb], adv, sb).wait()
        pltpu.make_async_copy(ae_hbm.at[pl.ds(off, CHUNK)], aec, se).wait()

    def wait_rows(jj, S):
        pltpu.make_async_copy(xl_hbm.at[S[1]], S[0], S[9]).wait()

    def fire_q(jj, S):
        dtb, pc, wmc, sac, q = S[2], S[6], S[7], S[8], S[13]
        pltpu.make_async_copy(pc, den_acc.at[dtb], q).start(add=True)
        pltpu.make_async_copy(wmc, cnt_acc.at[dtb], q).start(add=True)
        pltpu.make_async_copy(sac, s_acc.at[dtb], q).start(add=True)

    def wait_q(jj, S):
        dtb, pc, wmc, sac, q = S[2], S[6], S[7], S[8], S[13]
        pltpu.make_async_copy(pc, den_acc.at[dtb], q).wait()
        pltpu.make_async_copy(wmc, cnt_acc.at[dtb], q).wait()
        pltpu.make_async_copy(sac, s_acc.at[dtb], q).wait()

    def fire_r(jj, S):
        pltpu.make_async_copy(S[0], out_acc.at[S[2]], S[14]).start(add=True)

    def wait_r(jj, S):
        pltpu.make_async_copy(S[0], out_acc.at[S[2]], S[14]).wait()

    pltpu.sync_copy(sd_hbm.at[wid], sd)

    z16 = jnp.zeros((16,), _F32)

    @pl.loop(0, 40)
    def _zero_zb(i):
        zb[pl.ds(i * 16, 16)] = z16

    @pl.loop(0, CHUNK)
    def _zero_rows(r):
        for k in range(8):
            rows0[r, pl.ds(k * 16, 16)] = z16

    pltpu.sync_copy(zb, den_acc.at[pl.ds(base, 640)])
    pltpu.sync_copy(zb, s_acc.at[pl.ds(base, 640)])
    pltpu.sync_copy(zb, cnt_acc.at[pl.ds(base, 640)])

    @pl.loop(0, 640 // CHUNK)
    def _zero_out(i):
        pltpu.sync_copy(rows0, out_acc.at[pl.ds(base + i * CHUNK, CHUNK), :])

    plsc.subcore_barrier()

    neg = jnp.full((16,), -1e9, _F32)
    zero = jnp.zeros((16,), _F32)
    one = jnp.ones((16,), _F32)

    unpack(0, sets[0])
    gfire(0, sets[0])

    @pl.loop(0, CHUNKS, step=2)
    def _edges(j):
        for b in (0, 1):
            jj = j + b
            S = sets[b]
            T = sets[1 - b]
            nxt = jj + 1

            @pl.when(jnp.logical_and(nxt < CHUNKS, jj >= 1))
            def _wrq():
                wait_r(jj - 1, T)
                wait_q(jj - 1, T)

            @pl.when(nxt < CHUNKS)
            def _gf():
                unpack(nxt, T)
                gfire(nxt, T)

            wait_scalars(jj, S)

            rows, stb, dtb, asv_b, adv_b, aec, pc, wmc, sac = S[:9]
            for k in range(CHUNK // 16):
                sl = pl.ds(k * 16, 16)
                s16 = stb[sl]
                d16 = dtb[sl]
                ae16 = aec[sl]
                t = asv_b[sl] + adv_b[sl] + ae16
                nl = s16 != d16
                alpha = jnp.where(nl, t, neg)
                alpha = jnp.where(alpha > 0.0, alpha, alpha * 0.2)
                p = jnp.exp(alpha)
                w = jnp.where(nl, one, zero)
                pc[sl] = p
                wmc[sl] = w
                sac[sl] = ae16 * w

            fire_q(jj, S)

            wait_rows(jj, S)

            @pl.loop(0, CHUNK)
            def _scale(rr):
                ridx = jnp.full((16,), 0, jnp.int32) + rr
                pv = plsc.load_gather(pc, [ridx])
                for k in range(8):
                    sl2 = pl.ds(k * 16, 16)
                    rows[rr, sl2] = rows[rr, sl2] * pv

            fire_r(jj, S)

    wait_r(CHUNKS - 1, sets[1])
    wait_q(CHUNKS - 1, sets[1])

    plsc.subcore_barrier()

    pltpu.sync_copy(den_acc.at[pl.ds(base, 640)],
                    denp_hbm.at[cid, pl.ds(base, 640)])
    pltpu.sync_copy(s_acc.at[pl.ds(base, 640)],
                    sp_hbm.at[cid, pl.ds(base, 640)])
    pltpu.sync_copy(cnt_acc.at[pl.ds(base, 640)],
                    cntp_hbm.at[cid, pl.ds(base, 640)])

    @pl.loop(0, 640 // CHUNK)
    def _pub(i):
        pltpu.sync_copy(out_acc.at[pl.ds(base + i * CHUNK, CHUNK), :],
                        outp_hbm.at[cid, pl.ds(base + i * CHUNK, CHUNK), :])


def _sc_main(xl, a_src, a_dst, sd3, ae3):
    mesh = plsc.VectorSubcoreMesh(core_axis_name="c", subcore_axis_name="s")
    cp = pltpu.CompilerParams()
    if "needs_layout_passes" in pltpu.CompilerParams.__dataclass_fields__:
        cp = dataclasses.replace(cp, needs_layout_passes=False)
    kfn = pl.kernel(
        _sc_body,
        mesh=mesh,
        compiler_params=cp,
        out_type=[
            jax.ShapeDtypeStruct((2, N_PAD, C), _F32),
            jax.ShapeDtypeStruct((2, N_PAD), _F32),
            jax.ShapeDtypeStruct((2, N_PAD), _F32),
            jax.ShapeDtypeStruct((2, N_PAD), _F32),
        ],
        scratch_types=[
            pltpu.VMEM_SHARED((N_PAD, C), _F32),
            pltpu.VMEM_SHARED((N_PAD,), _F32),
            pltpu.VMEM_SHARED((N_PAD,), _F32),
            pltpu.VMEM_SHARED((N_PAD,), _F32),
            pltpu.VMEM((CHUNKS, CHUNK), jnp.int32),
            pltpu.VMEM((CHUNK, C), _F32),
            pltpu.VMEM((CHUNK, C), _F32),
            pltpu.VMEM((CHUNK,), jnp.int32),
            pltpu.VMEM((CHUNK,), jnp.int32),
            pltpu.VMEM((CHUNK,), jnp.int32),
            pltpu.VMEM((CHUNK,), jnp.int32),
            pltpu.VMEM((CHUNK,), _F32),
            pltpu.VMEM((CHUNK,), _F32),
            pltpu.VMEM((CHUNK,), _F32),
            pltpu.VMEM((CHUNK,), _F32),
            pltpu.VMEM((CHUNK,), _F32),
            pltpu.VMEM((CHUNK,), _F32),
            pltpu.VMEM((CHUNK,), _F32),
            pltpu.VMEM((CHUNK,), _F32),
            pltpu.VMEM((CHUNK,), _F32),
            pltpu.VMEM((CHUNK,), _F32),
            pltpu.VMEM((CHUNK,), _F32),
            pltpu.VMEM((CHUNK,), _F32),
            pltpu.VMEM((640,), _F32),
            pltpu.SemaphoreType.DMA,
            pltpu.SemaphoreType.DMA,
            pltpu.SemaphoreType.DMA,
            pltpu.SemaphoreType.DMA,
            pltpu.SemaphoreType.DMA,
            pltpu.SemaphoreType.DMA,
            pltpu.SemaphoreType.DMA,
            pltpu.SemaphoreType.DMA,
            pltpu.SemaphoreType.DMA,
            pltpu.SemaphoreType.DMA,
            pltpu.SemaphoreType.DMA,
            pltpu.SemaphoreType.DMA,
        ],
    )
    return kfn(xl, a_src, a_dst, sd3, ae3)


def _epi_body(outp_ref, denp_ref, sp_ref, cntp_ref, as_ref, ad_ref,
              xl_ref, b_ref, o_ref):
    den = denp_ref[0] + denp_ref[1]
    s = sp_ref[0] + sp_ref[1]
    cnt = cntp_ref[0] + cntp_ref[1]
    a_loop = s / jnp.maximum(cnt, 1.0)
    al = as_ref[...] + ad_ref[...] + a_loop
    al = jnp.where(al > 0.0, al, 0.2 * al)
    p_l = jnp.exp(al)
    outu = outp_ref[0] + outp_ref[1]
    o_ref[...] = (outu + p_l * xl_ref[...]) / (den + p_l + 1e-16) + b_ref[...]


_EPI_BLK = 1024


def _epilogue(outp, denp, sp, cntp, a_src, a_dst, xl, bias_row):
    nb = N_PAD // _EPI_BLK
    return pl.pallas_call(
        _epi_body,
        grid=(nb,),
        in_specs=[
            pl.BlockSpec((2, _EPI_BLK, C), lambda i: (0, i, 0)),
            pl.BlockSpec((2, _EPI_BLK, 1), lambda i: (0, i, 0)),
            pl.BlockSpec((2, _EPI_BLK, 1), lambda i: (0, i, 0)),
            pl.BlockSpec((2, _EPI_BLK, 1), lambda i: (0, i, 0)),
            pl.BlockSpec((_EPI_BLK, 1), lambda i: (i, 0)),
            pl.BlockSpec((_EPI_BLK, 1), lambda i: (i, 0)),
            pl.BlockSpec((_EPI_BLK, C), lambda i: (i, 0)),
            pl.BlockSpec((1, C), lambda i: (0, 0)),
        ],
        out_specs=pl.BlockSpec((_EPI_BLK, C), lambda i: (i, 0)),
        out_shape=jax.ShapeDtypeStruct((N_PAD, C), _F32),
    )(outp, denp, sp, cntp, a_src, a_dst, xl, bias_row)


def kernel(x, edge_index, edge_attr, W, att_src, att_dst, W_edge, att_edge,
           bias):
    x_pad = jnp.zeros((N_PAD, D_IN), _F32).at[:N].set(x)
    src = edge_index[0]
    dst = edge_index[1]
    pad = E_PAD - E
    pad_idx = jnp.arange(pad, dtype=jnp.int32) % N
    src_p = jnp.concatenate([src, pad_idx])
    dst_p = jnp.concatenate([dst, pad_idx])

    att_src_row = att_src.reshape(1, C)
    att_dst_row = att_dst.reshape(1, C)
    att_edge_row = att_edge.reshape(1, C)

    xl, a_src2, a_dst2 = _pre1(x_pad, W, att_src_row, att_dst_row)
    ae_row = _pre2(edge_attr.T, W_edge, att_edge_row)
    ae_flat = jnp.concatenate([ae_row.reshape(E), jnp.zeros((pad,), _F32)])

    sd3 = (src_p | (dst_p << 14)).reshape(N_TILES, CHUNKS, CHUNK)

    outp, denp, sp, cntp = _sc_main(
        xl, a_src2.reshape(N_PAD), a_dst2.reshape(N_PAD), sd3, ae_flat)

    out = _epilogue(outp,
                    denp.reshape(2, N_PAD, 1),
                    sp.reshape(2, N_PAD, 1),
                    cntp.reshape(2, N_PAD, 1),
                    a_src2, a_dst2, xl, bias.reshape(1, C))
    return out[:N]


# speedup vs baseline: 1.6756x; 1.1512x over previous
import dataclasses
import functools

import jax
import jax.numpy as jnp
from jax import lax
from jax.experimental import pallas as pl
from jax.experimental.pallas import tpu as pltpu
from jax.experimental.pallas import tpu_sc as plsc

N = 10000
E = 320000
D_IN = 128
D_EDGE = 16
C = 128

N_PAD = 10240
N_TILES = 32
EDGES_PER_TILE = 10240
CHUNKS = 128
CHUNK = 80
E_PAD = N_TILES * EDGES_PER_TILE

_F32 = jnp.float32


def _pre1_body(x_ref, w_ref, asv_ref, adv_ref, xl_ref, as_ref, ad_ref):
    xl = jnp.dot(x_ref[...], w_ref[...], preferred_element_type=_F32)
    xl_ref[...] = xl
    as_ref[...] = jnp.sum(xl * asv_ref[...], axis=1, keepdims=True)
    ad_ref[...] = jnp.sum(xl * adv_ref[...], axis=1, keepdims=True)


def _pre1(x_pad, w, att_src_row, att_dst_row):
    return pl.pallas_call(
        _pre1_body,
        out_shape=[
            jax.ShapeDtypeStruct((N_PAD, C), _F32),
            jax.ShapeDtypeStruct((N_PAD, 1), _F32),
            jax.ShapeDtypeStruct((N_PAD, 1), _F32),
        ],
    )(x_pad, w, att_src_row, att_dst_row)


_AE_BLK = 32000


def _pre2_body(ea_ref, we_ref, aev_ref, ae_ref):
    ve = jnp.sum(we_ref[...] * aev_ref[...], axis=1, keepdims=True)
    ae_ref[...] = jnp.sum(ea_ref[...] * ve, axis=0, keepdims=True)


def _pre2(ea_t, w_edge, att_edge_row):
    n = ea_t.shape[1]
    return pl.pallas_call(
        _pre2_body,
        grid=(n // _AE_BLK,),
        in_specs=[
            pl.BlockSpec((D_EDGE, _AE_BLK), lambda i: (0, i)),
            pl.BlockSpec((D_EDGE, C), lambda i: (0, 0)),
            pl.BlockSpec((1, C), lambda i: (0, 0)),
        ],
        out_specs=pl.BlockSpec((1, _AE_BLK), lambda i: (0, i)),
        out_shape=jax.ShapeDtypeStruct((1, n), _F32),
    )(ea_t, w_edge, att_edge_row)


def _sc_body(xl_hbm, asrc_hbm, adst_hbm, sd_hbm, ae_hbm,
             outp_hbm, denp_hbm, sp_hbm, cntp_hbm,
             out_acc, den_acc, s_acc, cnt_acc,
             sd,
             rows0, rows1, stb0, stb1, dtb0, dtb1,
             asv0, asv1, adv0, adv1, aec0, aec1,
             pc0, pc1, wmc0, wmc1, sac0, sac1, zb,
             g0, g1, sa0, sa1, sb0, sb1, se0, se1, q0, q1, r0, r1):
    cid = lax.axis_index("c")
    sid = lax.axis_index("s")
    wid = cid * 16 + sid
    base = sid * 640

    sets = (
        (rows0, stb0, dtb0, asv0, adv0, aec0, pc0, wmc0, sac0,
         g0, sa0, sb0, se0, q0, r0),
        (rows1, stb1, dtb1, asv1, adv1, aec1, pc1, wmc1, sac1,
         g1, sa1, sb1, se1, q1, r1),
    )

    mask14 = jnp.full((16,), 0x3FFF, jnp.int32)
    sh14 = jnp.full((16,), 14, jnp.int32)

    def unpack(jj, S):
        stb, dtb = S[1], S[2]
        for k in range(CHUNK // 16):
            sl = pl.ds(k * 16, 16)
            pk = sd[jj, sl]
            stb[sl] = pk & mask14
            dtb[sl] = lax.shift_right_logical(pk, sh14)

    def gfire(jj, S):
        rows, stb, dtb, asv, adv, aec = S[:6]
        g, sa, sb, se = S[9:13]
        off = wid * EDGES_PER_TILE + jj * CHUNK
        pltpu.async_copy(xl_hbm.at[stb], rows, g)
        pltpu.async_copy(asrc_hbm.at[stb], asv, sa)
        pltpu.async_copy(adst_hbm.at[dtb], adv, sb)
        pltpu.async_copy(ae_hbm.at[pl.ds(off, CHUNK)], aec, se)

    def wait_scalars(jj, S):
        rows, stb, dtb, asv, adv, aec = S[:6]
        g, sa, sb, se = S[9:13]
        off = wid * EDGES_PER_TILE + jj * CHUNK
        pltpu.make_async_copy(asrc_hbm.at[stb], asv, sa).wait()
        pltpu.make_async_copy(adst_hbm.at[dtb], adv, sb).wait()
        pltpu.make_async_copy(ae_hbm.at[pl.ds(off, CHUNK)], aec, se).wait()

    def wait_rows(jj, S):
        pltpu.make_async_copy(xl_hbm.at[S[1]], S[0], S[9]).wait()

    def fire_q(jj, S):
        dtb, pc, wmc, sac, q = S[2], S[6], S[7], S[8], S[13]
        pltpu.make_async_copy(pc, den_acc.at[dtb], q).start(add=True)
        pltpu.make_async_copy(wmc, cnt_acc.at[dtb], q).start(add=True)
        pltpu.make_async_copy(sac, s_acc.at[dtb], q).start(add=True)

    def wait_q(jj, S):
        dtb, pc, wmc, sac, q = S[2], S[6], S[7], S[8], S[13]
        pltpu.make_async_copy(pc, den_acc.at[dtb], q).wait()
        pltpu.make_async_copy(wmc, cnt_acc.at[dtb], q).wait()
        pltpu.make_async_copy(sac, s_acc.at[dtb], q).wait()

    def fire_r(jj, S):
        pltpu.make_async_copy(S[0], out_acc.at[S[2]], S[14]).start(add=True)

    def wait_r(jj, S):
        pltpu.make_async_copy(S[0], out_acc.at[S[2]], S[14]).wait()

    pltpu.sync_copy(sd_hbm.at[wid], sd)

    z16 = jnp.zeros((16,), _F32)

    @pl.loop(0, 40)
    def _zero_zb(i):
        zb[pl.ds(i * 16, 16)] = z16

    @pl.loop(0, CHUNK)
    def _zero_rows(r):
        for k in range(8):
            rows0[r, pl.ds(k * 16, 16)] = z16

    pltpu.sync_copy(zb, den_acc.at[pl.ds(base, 640)])
    pltpu.sync_copy(zb, s_acc.at[pl.ds(base, 640)])
    pltpu.sync_copy(zb, cnt_acc.at[pl.ds(base, 640)])

    @pl.loop(0, 640 // CHUNK)
    def _zero_out(i):
        pltpu.sync_copy(rows0, out_acc.at[pl.ds(base + i * CHUNK, CHUNK), :])

    plsc.subcore_barrier()

    neg = jnp.full((16,), -1e9, _F32)
    zero = jnp.zeros((16,), _F32)
    one = jnp.ones((16,), _F32)

    unpack(0, sets[0])
    gfire(0, sets[0])

    @pl.loop(0, CHUNKS, step=2)
    def _edges(j):
        for b in (0, 1):
            jj = j + b
            S = sets[b]
            T = sets[1 - b]
            nxt = jj + 1

            @pl.when(jnp.logical_and(nxt < CHUNKS, jj >= 1))
            def _wrq():
                wait_r(jj - 1, T)
                wait_q(jj - 1, T)

            @pl.when(nxt < CHUNKS)
            def _gf():
                unpack(nxt, T)
                gfire(nxt, T)

            wait_scalars(jj, S)

            rows, stb, dtb, asv_b, adv_b, aec, pc, wmc, sac = S[:9]
            for k in range(CHUNK // 16):
                sl = pl.ds(k * 16, 16)
                s16 = stb[sl]
                d16 = dtb[sl]
                ae16 = aec[sl]
                t = asv_b[sl] + adv_b[sl] + ae16
                nl = s16 != d16
                alpha = jnp.where(nl, t, neg)
                alpha = jnp.where(alpha > 0.0, alpha, alpha * 0.2)
                p = jnp.exp(alpha)
                w = jnp.where(nl, one, zero)
                pc[sl] = p
                wmc[sl] = w
                sac[sl] = ae16 * w

            fire_q(jj, S)

            wait_rows(jj, S)

            @pl.loop(0, CHUNK)
            def _scale(rr):
                ridx = jnp.full((16,), 0, jnp.int32) + rr
                pv = plsc.load_gather(pc, [ridx])
                for k in range(8):
                    sl2 = pl.ds(k * 16, 16)
                    rows[rr, sl2] = rows[rr, sl2] * pv

            fire_r(jj, S)

    wait_r(CHUNKS - 1, sets[1])
    wait_q(CHUNKS - 1, sets[1])

    plsc.subcore_barrier()

    pltpu.sync_copy(den_acc.at[pl.ds(base, 640)],
                    denp_hbm.at[cid, pl.ds(base, 640)])
    pltpu.sync_copy(s_acc.at[pl.ds(base, 640)],
                    sp_hbm.at[cid, pl.ds(base, 640)])
    pltpu.sync_copy(cnt_acc.at[pl.ds(base, 640)],
                    cntp_hbm.at[cid, pl.ds(base, 640)])

    @pl.loop(0, 640 // CHUNK)
    def _pub(i):
        pltpu.sync_copy(out_acc.at[pl.ds(base + i * CHUNK, CHUNK), :],
                        outp_hbm.at[cid, pl.ds(base + i * CHUNK, CHUNK), :])


def _sc_main(xl, a_src, a_dst, sd3, ae3):
    mesh = plsc.VectorSubcoreMesh(core_axis_name="c", subcore_axis_name="s")
    cp = pltpu.CompilerParams()
    if "needs_layout_passes" in pltpu.CompilerParams.__dataclass_fields__:
        cp = dataclasses.replace(cp, needs_layout_passes=False)
    kfn = pl.kernel(
        _sc_body,
        mesh=mesh,
        compiler_params=cp,
        out_type=[
            jax.ShapeDtypeStruct((2, N_PAD, C), _F32),
            jax.ShapeDtypeStruct((2, N_PAD), _F32),
            jax.ShapeDtypeStruct((2, N_PAD), _F32),
            jax.ShapeDtypeStruct((2, N_PAD), _F32),
        ],
        scratch_types=[
            pltpu.VMEM_SHARED((N_PAD, C), _F32),
            pltpu.VMEM_SHARED((N_PAD,), _F32),
            pltpu.VMEM_SHARED((N_PAD,), _F32),
            pltpu.VMEM_SHARED((N_PAD,), _F32),
            pltpu.VMEM((CHUNKS, CHUNK), jnp.int32),
            pltpu.VMEM((CHUNK, C), _F32),
            pltpu.VMEM((CHUNK, C), _F32),
            pltpu.VMEM((CHUNK,), jnp.int32),
            pltpu.VMEM((CHUNK,), jnp.int32),
            pltpu.VMEM((CHUNK,), jnp.int32),
            pltpu.VMEM((CHUNK,), jnp.int32),
            pltpu.VMEM((CHUNK,), _F32),
            pltpu.VMEM((CHUNK,), _F32),
            pltpu.VMEM((CHUNK,), _F32),
            pltpu.VMEM((CHUNK,), _F32),
            pltpu.VMEM((CHUNK,), _F32),
            pltpu.VMEM((CHUNK,), _F32),
            pltpu.VMEM((CHUNK,), _F32),
            pltpu.VMEM((CHUNK,), _F32),
            pltpu.VMEM((CHUNK,), _F32),
            pltpu.VMEM((CHUNK,), _F32),
            pltpu.VMEM((CHUNK,), _F32),
            pltpu.VMEM((CHUNK,), _F32),
            pltpu.VMEM((640,), _F32),
            pltpu.SemaphoreType.DMA,
            pltpu.SemaphoreType.DMA,
            pltpu.SemaphoreType.DMA,
            pltpu.SemaphoreType.DMA,
            pltpu.SemaphoreType.DMA,
            pltpu.SemaphoreType.DMA,
            pltpu.SemaphoreType.DMA,
            pltpu.SemaphoreType.DMA,
            pltpu.SemaphoreType.DMA,
            pltpu.SemaphoreType.DMA,
            pltpu.SemaphoreType.DMA,
            pltpu.SemaphoreType.DMA,
        ],
    )
    return kfn(xl, a_src, a_dst, sd3, ae3)


def _epi_body(outp_ref, denp_ref, sp_ref, cntp_ref, as_ref, ad_ref,
              xl_ref, b_ref, o_ref):
    den = (denp_ref[0] + denp_ref[1]).reshape(_EPI_BLK, 1)
    s = (sp_ref[0] + sp_ref[1]).reshape(_EPI_BLK, 1)
    cnt = (cntp_ref[0] + cntp_ref[1]).reshape(_EPI_BLK, 1)
    a_loop = s / jnp.maximum(cnt, 1.0)
    al = as_ref[...] + ad_ref[...] + a_loop
    al = jnp.where(al > 0.0, al, 0.2 * al)
    p_l = jnp.exp(al)
    outu = outp_ref[0] + outp_ref[1]
    o_ref[...] = (outu + p_l * xl_ref[...]) / (den + p_l + 1e-16) + b_ref[...]


_EPI_BLK = 1024


def _epilogue(outp, denp, sp, cntp, a_src, a_dst, xl, bias_row):
    nb = N_PAD // _EPI_BLK
    return pl.pallas_call(
        _epi_body,
        grid=(nb,),
        in_specs=[
            pl.BlockSpec((2, _EPI_BLK, C), lambda i: (0, i, 0)),
            pl.BlockSpec((2, _EPI_BLK), lambda i: (0, i)),
            pl.BlockSpec((2, _EPI_BLK), lambda i: (0, i)),
            pl.BlockSpec((2, _EPI_BLK), lambda i: (0, i)),
            pl.BlockSpec((_EPI_BLK, 1), lambda i: (i, 0)),
            pl.BlockSpec((_EPI_BLK, 1), lambda i: (i, 0)),
            pl.BlockSpec((_EPI_BLK, C), lambda i: (i, 0)),
            pl.BlockSpec((1, C), lambda i: (0, 0)),
        ],
        out_specs=pl.BlockSpec((_EPI_BLK, C), lambda i: (i, 0)),
        out_shape=jax.ShapeDtypeStruct((N_PAD, C), _F32),
    )(outp, denp, sp, cntp, a_src, a_dst, xl, bias_row)


def kernel(x, edge_index, edge_attr, W, att_src, att_dst, W_edge, att_edge,
           bias):
    x_pad = jnp.zeros((N_PAD, D_IN), _F32).at[:N].set(x)
    src = edge_index[0]
    dst = edge_index[1]
    pad = E_PAD - E
    pad_idx = jnp.arange(pad, dtype=jnp.int32) % N
    src_p = jnp.concatenate([src, pad_idx])
    dst_p = jnp.concatenate([dst, pad_idx])

    att_src_row = att_src.reshape(1, C)
    att_dst_row = att_dst.reshape(1, C)
    att_edge_row = att_edge.reshape(1, C)

    xl, a_src2, a_dst2 = _pre1(x_pad, W, att_src_row, att_dst_row)
    ae_row = _pre2(edge_attr.T, W_edge, att_edge_row)
    ae_flat = jnp.concatenate([ae_row.reshape(E), jnp.zeros((pad,), _F32)])

    sd3 = (src_p | (dst_p << 14)).reshape(N_TILES, CHUNKS, CHUNK)

    outp, denp, sp, cntp = _sc_main(
        xl, a_src2.reshape(N_PAD), a_dst2.reshape(N_PAD), sd3, ae_flat)

    out = _epilogue(outp, denp, sp, cntp,
                    a_src2, a_dst2, xl, bias.reshape(1, C))
    return out[:N]


# speedup vs baseline: 1.7052x; 1.0177x over previous
import dataclasses
import functools

import jax
import jax.numpy as jnp
from jax import lax
from jax.experimental import pallas as pl
from jax.experimental.pallas import tpu as pltpu
from jax.experimental.pallas import tpu_sc as plsc

N = 10000
E = 320000
D_IN = 128
D_EDGE = 16
C = 128

N_PAD = 10240
N_TILES = 32
EDGES_PER_TILE = 10240
CHUNKS = 128
CHUNK = 80
E_PAD = N_TILES * EDGES_PER_TILE

_F32 = jnp.float32


def _pre1_body(x_ref, w_ref, asv_ref, adv_ref, xl_ref, as_ref, ad_ref):
    xl = jnp.dot(x_ref[...], w_ref[...], preferred_element_type=_F32)
    xl_ref[...] = xl
    as_ref[...] = jnp.sum(xl * asv_ref[...], axis=1, keepdims=True)
    ad_ref[...] = jnp.sum(xl * adv_ref[...], axis=1, keepdims=True)


_P1_BLK = 2560


def _pre1(x_pad, w, att_src_row, att_dst_row):
    nb = N_PAD // _P1_BLK
    return pl.pallas_call(
        _pre1_body,
        grid=(nb,),
        in_specs=[
            pl.BlockSpec((_P1_BLK, D_IN), lambda i: (i, 0)),
            pl.BlockSpec((D_IN, C), lambda i: (0, 0)),
            pl.BlockSpec((1, C), lambda i: (0, 0)),
            pl.BlockSpec((1, C), lambda i: (0, 0)),
        ],
        out_specs=[
            pl.BlockSpec((_P1_BLK, C), lambda i: (i, 0)),
            pl.BlockSpec((_P1_BLK, 1), lambda i: (i, 0)),
            pl.BlockSpec((_P1_BLK, 1), lambda i: (i, 0)),
        ],
        out_shape=[
            jax.ShapeDtypeStruct((N_PAD, C), _F32),
            jax.ShapeDtypeStruct((N_PAD, 1), _F32),
            jax.ShapeDtypeStruct((N_PAD, 1), _F32),
        ],
    )(x_pad, w, att_src_row, att_dst_row)


_AE_BLK = 32000


def _pre2_body(ea_ref, we_ref, aev_ref, ae_ref):
    ve = jnp.sum(we_ref[...] * aev_ref[...], axis=1, keepdims=True)
    ae_ref[...] = jnp.sum(ea_ref[...] * ve, axis=0, keepdims=True)


def _pre2(ea_t, w_edge, att_edge_row):
    n = ea_t.shape[1]
    return pl.pallas_call(
        _pre2_body,
        grid=(n // _AE_BLK,),
        in_specs=[
            pl.BlockSpec((D_EDGE, _AE_BLK), lambda i: (0, i)),
            pl.BlockSpec((D_EDGE, C), lambda i: (0, 0)),
            pl.BlockSpec((1, C), lambda i: (0, 0)),
        ],
        out_specs=pl.BlockSpec((1, _AE_BLK), lambda i: (0, i)),
        out_shape=jax.ShapeDtypeStruct((1, n), _F32),
    )(ea_t, w_edge, att_edge_row)


def _sc_body(xl_hbm, asrc_hbm, adst_hbm, sd_hbm, ae_hbm,
             outp_hbm, denp_hbm, sp_hbm, cntp_hbm,
             out_acc, den_acc, s_acc, cnt_acc,
             sd,
             rows0, rows1, stb0, stb1, dtb0, dtb1,
             asv0, asv1, adv0, adv1, aec0, aec1,
             pc0, pc1, wmc0, wmc1, sac0, sac1, zb,
             g0, g1, sa0, sa1, sb0, sb1, se0, se1, q0, q1, r0, r1):
    cid = lax.axis_index("c")
    sid = lax.axis_index("s")
    wid = cid * 16 + sid
    base = sid * 640

    sets = (
        (rows0, stb0, dtb0, asv0, adv0, aec0, pc0, wmc0, sac0,
         g0, sa0, sb0, se0, q0, r0),
        (rows1, stb1, dtb1, asv1, adv1, aec1, pc1, wmc1, sac1,
         g1, sa1, sb1, se1, q1, r1),
    )

    mask14 = jnp.full((16,), 0x3FFF, jnp.int32)
    sh14 = jnp.full((16,), 14, jnp.int32)

    def unpack(jj, S):
        stb, dtb = S[1], S[2]
        for k in range(CHUNK // 16):
            sl = pl.ds(k * 16, 16)
            pk = sd[jj, sl]
            stb[sl] = pk & mask14
            dtb[sl] = lax.shift_right_logical(pk, sh14)

    def gfire(jj, S):
        rows, stb, dtb, asv, adv, aec = S[:6]
        g, sa, sb, se = S[9:13]
        off = wid * EDGES_PER_TILE + jj * CHUNK
        pltpu.async_copy(xl_hbm.at[stb], rows, g)
        pltpu.async_copy(asrc_hbm.at[stb], asv, sa)
        pltpu.async_copy(adst_hbm.at[dtb], adv, sb)
        pltpu.async_copy(ae_hbm.at[pl.ds(off, CHUNK)], aec, se)

    def wait_scalars(jj, S):
        rows, stb, dtb, asv, adv, aec = S[:6]
        g, sa, sb, se = S[9:13]
        off = wid * EDGES_PER_TILE + jj * CHUNK
        pltpu.make_async_copy(asrc_hbm.at[stb], asv, sa).wait()
        pltpu.make_async_copy(adst_hbm.at[dtb], adv, sb).wait()
        pltpu.make_async_copy(ae_hbm.at[pl.ds(off, CHUNK)], aec, se).wait()

    def wait_rows(jj, S):
        pltpu.make_async_copy(xl_hbm.at[S[1]], S[0], S[9]).wait()

    def fire_q(jj, S):
        dtb, pc, wmc, sac, q = S[2], S[6], S[7], S[8], S[13]
        pltpu.make_async_copy(pc, den_acc.at[dtb], q).start(add=True)
        pltpu.make_async_copy(wmc, cnt_acc.at[dtb], q).start(add=True)
        pltpu.make_async_copy(sac, s_acc.at[dtb], q).start(add=True)

    def wait_q(jj, S):
        dtb, pc, wmc, sac, q = S[2], S[6], S[7], S[8], S[13]
        pltpu.make_async_copy(pc, den_acc.at[dtb], q).wait()
        pltpu.make_async_copy(wmc, cnt_acc.at[dtb], q).wait()
        pltpu.make_async_copy(sac, s_acc.at[dtb], q).wait()

    def fire_r(jj, S):
        pltpu.make_async_copy(S[0], out_acc.at[S[2]], S[14]).start(add=True)

    def wait_r(jj, S):
        pltpu.make_async_copy(S[0], out_acc.at[S[2]], S[14]).wait()

    pltpu.sync_copy(sd_hbm.at[wid], sd)

    z16 = jnp.zeros((16,), _F32)

    @pl.loop(0, 40)
    def _zero_zb(i):
        zb[pl.ds(i * 16, 16)] = z16

    @pl.loop(0, CHUNK)
    def _zero_rows(r):
        for k in range(8):
            rows0[r, pl.ds(k * 16, 16)] = z16

    pltpu.sync_copy(zb, den_acc.at[pl.ds(base, 640)])
    pltpu.sync_copy(zb, s_acc.at[pl.ds(base, 640)])
    pltpu.sync_copy(zb, cnt_acc.at[pl.ds(base, 640)])

    @pl.loop(0, 640 // CHUNK)
    def _zero_out(i):
        pltpu.sync_copy(rows0, out_acc.at[pl.ds(base + i * CHUNK, CHUNK), :])

    plsc.subcore_barrier()

    neg = jnp.full((16,), -1e9, _F32)
    zero = jnp.zeros((16,), _F32)
    one = jnp.ones((16,), _F32)

    unpack(0, sets[0])
    gfire(0, sets[0])

    @pl.loop(0, CHUNKS, step=2)
    def _edges(j):
        for b in (0, 1):
            jj = j + b
            S = sets[b]
            T = sets[1 - b]
            nxt = jj + 1

            @pl.when(jnp.logical_and(nxt < CHUNKS, jj >= 1))
            def _wrq():
                wait_r(jj - 1, T)
                wait_q(jj - 1, T)

            @pl.when(nxt < CHUNKS)
            def _gf():
                unpack(nxt, T)
                gfire(nxt, T)

            wait_scalars(jj, S)

            rows, stb, dtb, asv_b, adv_b, aec, pc, wmc, sac = S[:9]
            for k in range(CHUNK // 16):
                sl = pl.ds(k * 16, 16)
                s16 = stb[sl]
                d16 = dtb[sl]
                ae16 = aec[sl]
                t = asv_b[sl] + adv_b[sl] + ae16
                nl = s16 != d16
                alpha = jnp.where(nl, t, neg)
                alpha = jnp.where(alpha > 0.0, alpha, alpha * 0.2)
                p = jnp.exp(alpha)
                w = jnp.where(nl, one, zero)
                pc[sl] = p
                wmc[sl] = w
                sac[sl] = ae16 * w

            fire_q(jj, S)

            wait_rows(jj, S)

            @pl.loop(0, CHUNK, step=2)
            def _scale(rr):
                for u in range(2):
                    r2 = rr + u
                    ridx = jnp.full((16,), 0, jnp.int32) + r2
                    pv = plsc.load_gather(pc, [ridx])
                    for k in range(8):
                        sl2 = pl.ds(k * 16, 16)
                        rows[r2, sl2] = rows[r2, sl2] * pv

            fire_r(jj, S)

    wait_r(CHUNKS - 1, sets[1])
    wait_q(CHUNKS - 1, sets[1])

    plsc.subcore_barrier()

    pltpu.sync_copy(den_acc.at[pl.ds(base, 640)],
                    denp_hbm.at[cid, pl.ds(base, 640)])
    pltpu.sync_copy(s_acc.at[pl.ds(base, 640)],
                    sp_hbm.at[cid, pl.ds(base, 640)])
    pltpu.sync_copy(cnt_acc.at[pl.ds(base, 640)],
                    cntp_hbm.at[cid, pl.ds(base, 640)])

    @pl.loop(0, 640 // CHUNK)
    def _pub(i):
        pltpu.sync_copy(out_acc.at[pl.ds(base + i * CHUNK, CHUNK), :],
                        outp_hbm.at[cid, pl.ds(base + i * CHUNK, CHUNK), :])


def _sc_main(xl, a_src, a_dst, sd3, ae3):
    mesh = plsc.VectorSubcoreMesh(core_axis_name="c", subcore_axis_name="s")
    cp = pltpu.CompilerParams()
    if "needs_layout_passes" in pltpu.CompilerParams.__dataclass_fields__:
        cp = dataclasses.replace(cp, needs_layout_passes=False)
    kfn = pl.kernel(
        _sc_body,
        mesh=mesh,
        compiler_params=cp,
        out_type=[
            jax.ShapeDtypeStruct((2, N_PAD, C), _F32),
            jax.ShapeDtypeStruct((2, N_PAD), _F32),
            jax.ShapeDtypeStruct((2, N_PAD), _F32),
            jax.ShapeDtypeStruct((2, N_PAD), _F32),
        ],
        scratch_types=[
            pltpu.VMEM_SHARED((N_PAD, C), _F32),
            pltpu.VMEM_SHARED((N_PAD,), _F32),
            pltpu.VMEM_SHARED((N_PAD,), _F32),
            pltpu.VMEM_SHARED((N_PAD,), _F32),
            pltpu.VMEM((CHUNKS, CHUNK), jnp.int32),
            pltpu.VMEM((CHUNK, C), _F32),
            pltpu.VMEM((CHUNK, C), _F32),
            pltpu.VMEM((CHUNK,), jnp.int32),
            pltpu.VMEM((CHUNK,), jnp.int32),
            pltpu.VMEM((CHUNK,), jnp.int32),
            pltpu.VMEM((CHUNK,), jnp.int32),
            pltpu.VMEM((CHUNK,), _F32),
            pltpu.VMEM((CHUNK,), _F32),
            pltpu.VMEM((CHUNK,), _F32),
            pltpu.VMEM((CHUNK,), _F32),
            pltpu.VMEM((CHUNK,), _F32),
            pltpu.VMEM((CHUNK,), _F32),
            pltpu.VMEM((CHUNK,), _F32),
            pltpu.VMEM((CHUNK,), _F32),
            pltpu.VMEM((CHUNK,), _F32),
            pltpu.VMEM((CHUNK,), _F32),
            pltpu.VMEM((CHUNK,), _F32),
            pltpu.VMEM((CHUNK,), _F32),
            pltpu.VMEM((640,), _F32),
            pltpu.SemaphoreType.DMA,
            pltpu.SemaphoreType.DMA,
            pltpu.SemaphoreType.DMA,
            pltpu.SemaphoreType.DMA,
            pltpu.SemaphoreType.DMA,
            pltpu.SemaphoreType.DMA,
            pltpu.SemaphoreType.DMA,
            pltpu.SemaphoreType.DMA,
            pltpu.SemaphoreType.DMA,
            pltpu.SemaphoreType.DMA,
            pltpu.SemaphoreType.DMA,
            pltpu.SemaphoreType.DMA,
        ],
    )
    return kfn(xl, a_src, a_dst, sd3, ae3)


def _epi_body(outp_ref, denp_ref, sp_ref, cntp_ref, as_ref, ad_ref,
              xl_ref, b_ref, o_ref):
    den = (denp_ref[0] + denp_ref[1]).reshape(_EPI_BLK, 1)
    s = (sp_ref[0] + sp_ref[1]).reshape(_EPI_BLK, 1)
    cnt = (cntp_ref[0] + cntp_ref[1]).reshape(_EPI_BLK, 1)
    a_loop = s / jnp.maximum(cnt, 1.0)
    al = as_ref[...] + ad_ref[...] + a_loop
    al = jnp.where(al > 0.0, al, 0.2 * al)
    p_l = jnp.exp(al)
    outu = outp_ref[0] + outp_ref[1]
    o_ref[...] = (outu + p_l * xl_ref[...]) / (den + p_l + 1e-16) + b_ref[...]


_EPI_BLK = 1024


def _epilogue(outp, denp, sp, cntp, a_src, a_dst, xl, bias_row):
    nb = N_PAD // _EPI_BLK
    return pl.pallas_call(
        _epi_body,
        grid=(nb,),
        in_specs=[
            pl.BlockSpec((2, _EPI_BLK, C), lambda i: (0, i, 0)),
            pl.BlockSpec((2, _EPI_BLK), lambda i: (0, i)),
            pl.BlockSpec((2, _EPI_BLK), lambda i: (0, i)),
            pl.BlockSpec((2, _EPI_BLK), lambda i: (0, i)),
            pl.BlockSpec((_EPI_BLK, 1), lambda i: (i, 0)),
            pl.BlockSpec((_EPI_BLK, 1), lambda i: (i, 0)),
            pl.BlockSpec((_EPI_BLK, C), lambda i: (i, 0)),
            pl.BlockSpec((1, C), lambda i: (0, 0)),
        ],
        out_specs=pl.BlockSpec((_EPI_BLK, C), lambda i: (i, 0)),
        out_shape=jax.ShapeDtypeStruct((N_PAD, C), _F32),
    )(outp, denp, sp, cntp, a_src, a_dst, xl, bias_row)


def kernel(x, edge_index, edge_attr, W, att_src, att_dst, W_edge, att_edge,
           bias):
    x_pad = jnp.zeros((N_PAD, D_IN), _F32).at[:N].set(x)
    src = edge_index[0]
    dst = edge_index[1]
    pad = E_PAD - E
    pad_idx = jnp.arange(pad, dtype=jnp.int32) % N
    src_p = jnp.concatenate([src, pad_idx])
    dst_p = jnp.concatenate([dst, pad_idx])

    att_src_row = att_src.reshape(1, C)
    att_dst_row = att_dst.reshape(1, C)
    att_edge_row = att_edge.reshape(1, C)

    xl, a_src2, a_dst2 = _pre1(x_pad, W, att_src_row, att_dst_row)
    ae_row = _pre2(edge_attr.T, W_edge, att_edge_row)
    ae_flat = jnp.concatenate([ae_row.reshape(E), jnp.zeros((pad,), _F32)])

    sd3 = (src_p | (dst_p << 14)).reshape(N_TILES, CHUNKS, CHUNK)

    outp, denp, sp, cntp = _sc_main(
        xl, a_src2.reshape(N_PAD), a_dst2.reshape(N_PAD), sd3, ae_flat)

    out = _epilogue(outp, denp, sp, cntp,
                    a_src2, a_dst2, xl, bias.reshape(1, C))
    return out[:N]
